# baseline (speedup 1.0000x reference)
import functools

import jax
import jax.numpy as jnp
from jax import lax
from jax.experimental import pallas as pl
from jax.experimental.pallas import tpu as pltpu
from jax.experimental.pallas import tpu_sc as plsc

_ALPHA = 32.0
_BLK = 256
_CH = 64


def _router_body(x_ref, rw_ref, la_ref, lb_ref,
                 oh1_ref, oh2_ref, w1_ref, w2_ref, lora_ref, xbf_ref):
    x = x_ref[...]
    logits = jnp.dot(x, rw_ref[...], preferred_element_type=jnp.float32)
    m = jnp.max(logits, axis=-1, keepdims=True)
    p = jnp.exp(logits - m)
    p = p / jnp.sum(p, axis=-1, keepdims=True)
    eidx = jax.lax.broadcasted_iota(jnp.int32, p.shape, 1)
    i1 = jnp.argmax(p, axis=-1)
    oh1 = (eidx == i1[:, None]).astype(jnp.float32)
    v1 = jnp.max(p, axis=-1)
    pm = jnp.where(oh1 > 0, -jnp.inf, p)
    i2 = jnp.argmax(pm, axis=-1)
    oh2 = (eidx == i2[:, None]).astype(jnp.float32)
    v2 = jnp.max(pm, axis=-1)
    den = v1 + v2
    oh1_ref[...] = oh1
    oh2_ref[...] = oh2
    w1_ref[...] = (v1 / den)[:, None]
    w2_ref[...] = (v2 / den)[:, None]
    r = la_ref.shape[1]
    lora = jnp.dot(jnp.dot(x, la_ref[...], preferred_element_type=jnp.float32),
                   lb_ref[...], preferred_element_type=jnp.float32)
    lora_ref[...] = lora * (_ALPHA / r)
    xbf_ref[...] = x.astype(jnp.bfloat16)


def _router(x, router_w, lora_A, lora_B):
    t, h = x.shape
    e = router_w.shape[1]
    r = lora_A.shape[1]
    tb = 512
    return pl.pallas_call(
        _router_body,
        grid=(t // tb,),
        in_specs=[
            pl.BlockSpec((tb, h), lambda i: (i, 0)),
            pl.BlockSpec((h, e), lambda i: (0, 0)),
            pl.BlockSpec((h, r), lambda i: (0, 0)),
            pl.BlockSpec((r, h), lambda i: (0, 0)),
        ],
        out_specs=[
            pl.BlockSpec((tb, e), lambda i: (i, 0)),
            pl.BlockSpec((tb, e), lambda i: (i, 0)),
            pl.BlockSpec((tb, 1), lambda i: (i, 0)),
            pl.BlockSpec((tb, 1), lambda i: (i, 0)),
            pl.BlockSpec((tb, h), lambda i: (i, 0)),
            pl.BlockSpec((tb, h), lambda i: (i, 0)),
        ],
        out_shape=[
            jax.ShapeDtypeStruct((t, e), jnp.float32),
            jax.ShapeDtypeStruct((t, e), jnp.float32),
            jax.ShapeDtypeStruct((t, 1), jnp.float32),
            jax.ShapeDtypeStruct((t, 1), jnp.float32),
            jax.ShapeDtypeStruct((t, h), jnp.float32),
            jax.ShapeDtypeStruct((t, h), jnp.bfloat16),
        ],
    )(x, router_w, lora_A, lora_B)


def _dispatch_body(oh1_ref, oh2_ref, pos_ref, be_ref, rank_ref):
    t, e = oh1_ref.shape
    npairs = 2 * t
    nchunks = npairs // _BLK
    half = nchunks // 2
    nb_total = pos_ref.shape[0] // _BLK + e

    rows = jax.lax.broadcasted_iota(jnp.int32, (_BLK, _BLK), 0)
    cols = jax.lax.broadcasted_iota(jnp.int32, (_BLK, _BLK), 1)
    lts = (rows > cols).astype(jnp.float32)

    def load_chunk(j):
        o = (lax.rem(j, half)) * _BLK
        a = oh1_ref[pl.ds(o, _BLK), :]
        b = oh2_ref[pl.ds(o, _BLK), :]
        return jnp.where(j < half, a, b)

    def pass1(j, carry):
        oh = load_chunk(j)
        rank = jnp.dot(lts, oh, preferred_element_type=jnp.float32) + carry
        rank_ref[pl.ds(j * _BLK, _BLK), :] = rank
        return carry + jnp.sum(oh, axis=0, keepdims=True)

    cnt = lax.fori_loop(0, nchunks, pass1, jnp.zeros((1, e), jnp.float32))

    nb = jnp.floor((cnt + (_BLK - 1)) * (1.0 / _BLK))
    ri = jax.lax.broadcasted_iota(jnp.int32, (e, e), 0)
    ci = jax.lax.broadcasted_iota(jnp.int32, (e, e), 1)
    ut = (ri < ci).astype(jnp.float32)
    start_blk = jnp.dot(nb, ut, preferred_element_type=jnp.float32)
    off = start_blk * float(_BLK)

    def pass2(j, _):
        oh = load_chunk(j)
        rank = rank_ref[pl.ds(j * _BLK, _BLK), :]
        posf = jnp.sum(oh * (rank + off), axis=1, keepdims=True)
        pos_ref[pl.ds(j * _BLK, _BLK), :] = posf.astype(jnp.int32)
        return 0

    lax.fori_loop(0, nchunks, pass2, 0)

    bi = jax.lax.broadcasted_iota(jnp.int32, (nb_total, e), 0).astype(jnp.float32)
    be = jnp.sum((bi >= start_blk).astype(jnp.float32), axis=1, keepdims=True)
    be_ref[...] = be.astype(jnp.int32) - 1


def _dispatch(oh1, oh2, nb_total):
    t, e = oh1.shape
    return pl.pallas_call(
        _dispatch_body,
        out_shape=[
            jax.ShapeDtypeStruct((2 * t, 1), jnp.int32),
            jax.ShapeDtypeStruct((nb_total, 1), jnp.int32),
        ],
        scratch_shapes=[pltpu.VMEM((2 * t, e), jnp.float32)],
    )(oh1, oh2)


def _make_sc_scatter(t, hw, cap, nw):
    rows_per_w = 2 * t // nw
    nchunk = rows_per_w // _CH
    mesh = plsc.VectorSubcoreMesh(core_axis_name="c", subcore_axis_name="s",
                                  num_cores=2, num_subcores=16)

    @functools.partial(
        pl.kernel,
        out_type=(
            jax.ShapeDtypeStruct((cap, hw), jnp.int32),
            jax.ShapeDtypeStruct((cap,), jnp.float32),
        ),
        mesh=mesh,
        scratch_types=[
            pltpu.VMEM((nchunk, _CH), jnp.int32),
            pltpu.VMEM((nchunk, _CH), jnp.float32),
            pltpu.VMEM((2, _CH, hw), jnp.int32),
            pltpu.SemaphoreType.DMA,
            pltpu.SemaphoreType.DMA,
            pltpu.SemaphoreType.DMA,
            pltpu.SemaphoreType.DMA,
            pltpu.SemaphoreType.DMA,
        ],
    )
    def sc_scatter(pos2d, w2d, x, xs, wsort, idx_v, w_v, xbuf,
                   lsem0, lsem1, ssem0, ssem1, wsem):
        nc = 2
        wid = lax.axis_index("s") * nc + lax.axis_index("c")
        row0 = wid * nchunk
        pltpu.sync_copy(pos2d.at[pl.ds(row0, nchunk)], idx_v)
        pltpu.sync_copy(w2d.at[pl.ds(row0, nchunk)], w_v)
        tbase = lax.rem(wid, nw // 2) * rows_per_w
        lsems = (lsem0, lsem1)
        ssems = (ssem0, ssem1)

        loads = [None] * nchunk
        stores = [None] * nchunk
        wstores = [None] * nchunk
        loads[0] = pltpu.async_copy(
            x.at[pl.ds(tbase, _CH)], xbuf.at[0], lsems[0])
        for c in range(nchunk):
            if c >= 1:
                stores[c - 1].wait()
            if c + 1 < nchunk:
                loads[c + 1] = pltpu.async_copy(
                    x.at[pl.ds(tbase + (c + 1) * _CH, _CH)],
                    xbuf.at[(c + 1) % 2], lsems[(c + 1) % 2])
            loads[c].wait()
            stores[c] = pltpu.async_copy(
                xbuf.at[c % 2], xs.at[idx_v.at[c]], ssems[c % 2])
            wstores[c] = pltpu.async_copy(
                w_v.at[c], wsort.at[idx_v.at[c]], wsem)
        stores[nchunk - 1].wait()
        for c in range(nchunk):
            wstores[c].wait()

    return sc_scatter


def _gmm_body(be_ref, xs_ref, ws_ref, wg_ref, wu_ref, wd_ref, out_ref):
    xb = xs_ref[...]
    g = jnp.dot(xb, wg_ref[0], preferred_element_type=jnp.float32)
    u = jnp.dot(xb, wu_ref[0], preferred_element_type=jnp.float32)
    hdn = (g * jax.nn.sigmoid(g)) * u * ws_ref[...]
    y = jnp.dot(hdn.astype(jnp.bfloat16), wd_ref[0],
                preferred_element_type=jnp.float32)
    out_ref[...] = y.astype(jnp.bfloat16)


def _gmm(xs, wsort, W_gate, W_up, W_down, be):
    cap, h = xs.shape
    e, _, esz = W_gate.shape
    nb_total = cap // _BLK
    grid_spec = pltpu.PrefetchScalarGridSpec(
        num_scalar_prefetch=1,
        grid=(nb_total,),
        in_specs=[
            pl.BlockSpec((_BLK, h), lambda i, s: (i, 0)),
            pl.BlockSpec((_BLK, 1), lambda i, s: (i, 0)),
            pl.BlockSpec((1, h, esz), lambda i, s: (s[i], 0, 0)),
            pl.BlockSpec((1, h, esz), lambda i, s: (s[i], 0, 0)),
            pl.BlockSpec((1, esz, h), lambda i, s: (s[i], 0, 0)),
        ],
        out_specs=pl.BlockSpec((_BLK, h), lambda i, s: (i, 0)),
    )
    return pl.pallas_call(
        _gmm_body,
        grid_spec=grid_spec,
        out_shape=jax.ShapeDtypeStruct((cap, h), jnp.bfloat16),
        compiler_params=pltpu.CompilerParams(
            dimension_semantics=("arbitrary",),
        ),
    )(be, xs, wsort, W_gate, W_up, W_down)


def _make_sc_gather(t, hw, cap, nw):
    pairs_per_w = 2 * t // nw
    nchunk = pairs_per_w // _CH
    mesh = plsc.VectorSubcoreMesh(core_axis_name="c", subcore_axis_name="s",
                                  num_cores=2, num_subcores=16)

    @functools.partial(
        pl.kernel,
        out_type=jax.ShapeDtypeStruct((2 * t, hw), jnp.int32),
        mesh=mesh,
        scratch_types=[
            pltpu.VMEM((nchunk, _CH), jnp.int32),
            pltpu.VMEM((2, _CH, hw), jnp.int32),
            pltpu.SemaphoreType.DMA,
            pltpu.SemaphoreType.DMA,
            pltpu.SemaphoreType.DMA,
            pltpu.SemaphoreType.DMA,
        ],
    )
    def sc_gather(ys, pos2d, ytcat, idx_v, buf, gsem0, gsem1, ssem0, ssem1):
        nc = 2
        wid = lax.axis_index("s") * nc + lax.axis_index("c")
        pltpu.sync_copy(pos2d.at[pl.ds(wid * nchunk, nchunk)], idx_v)
        p0 = wid * pairs_per_w
        gsems = (gsem0, gsem1)
        ssems = (ssem0, ssem1)

        gathers = [None] * nchunk
        stores = [None] * nchunk
        gathers[0] = pltpu.async_copy(ys.at[idx_v.at[0]], buf.at[0], gsems[0])
        for c in range(nchunk):
            if c >= 1:
                stores[c - 1].wait()
            if c + 1 < nchunk:
                gathers[c + 1] = pltpu.async_copy(
                    ys.at[idx_v.at[c + 1]], buf.at[(c + 1) % 2],
                    gsems[(c + 1) % 2])
            gathers[c].wait()
            stores[c] = pltpu.async_copy(
                buf.at[c % 2], ytcat.at[pl.ds(p0 + c * _CH, _CH)],
                ssems[c % 2])
        stores[nchunk - 1].wait()

    return sc_gather


def _finalize_body(lora_ref, y1_ref, y2_ref, out_ref):
    out_ref[...] = (lora_ref[...]
                    + y1_ref[...].astype(jnp.float32)
                    + y2_ref[...].astype(jnp.float32))


def _finalize(lora, ytcat):
    t, h = lora.shape
    tb = 512
    return pl.pallas_call(
        _finalize_body,
        grid=(t // tb,),
        in_specs=[
            pl.BlockSpec((tb, h), lambda i: (i, 0)),
            pl.BlockSpec((tb, h), lambda i: (i, 0)),
            pl.BlockSpec((tb, h), lambda i: (i + t // tb, 0)),
        ],
        out_specs=pl.BlockSpec((tb, h), lambda i: (i, 0)),
        out_shape=jax.ShapeDtypeStruct((t, h), jnp.float32),
    )(lora, ytcat, ytcat)


def kernel(input, router_w, W_gate, W_up, W_down, lora_A, lora_B):
    b, s, h = input.shape
    t = b * s
    e = router_w.shape[1]
    x = input.reshape(t, h)
    nb_total = (2 * t) // _BLK + e
    cap = nb_total * _BLK
    nw = 32
    hw = h // 2

    oh1, oh2, w1, w2, lora, xbf = _router(x, router_w, lora_A, lora_B)
    pos, be = _dispatch(oh1, oh2, nb_total)

    pos2d = pos.reshape(2 * t // _CH, _CH)
    w2d = jnp.concatenate([w1, w2], axis=0).reshape(2 * t // _CH, _CH)
    xbf_i32 = lax.bitcast_convert_type(xbf.reshape(t, hw, 2), jnp.int32)

    xs_i32, wsort = _make_sc_scatter(t, hw, cap, nw)(pos2d, w2d, xbf_i32)
    xs = lax.bitcast_convert_type(xs_i32, jnp.bfloat16).reshape(cap, h)
    ys = _gmm(xs, wsort.reshape(cap, 1),
              W_gate.astype(jnp.bfloat16), W_up.astype(jnp.bfloat16),
              W_down.astype(jnp.bfloat16), be.reshape(nb_total))
    ys_i32 = lax.bitcast_convert_type(ys.reshape(cap, hw, 2), jnp.int32)
    yt_i32 = _make_sc_gather(t, hw, cap, nw)(ys_i32, pos2d)
    ytcat = lax.bitcast_convert_type(yt_i32, jnp.bfloat16).reshape(2 * t, h)
    out = _finalize(lora, ytcat)
    return out.reshape(b, s, h)


# speedup vs baseline: 5.1209x; 5.1209x over previous
import functools

import jax
import jax.numpy as jnp
from jax import lax
from jax.experimental import pallas as pl
from jax.experimental.pallas import tpu as pltpu
from jax.experimental.pallas import tpu_sc as plsc

_ALPHA = 32.0
_BLK = 256
_CH = 64


def _router_body(x_ref, rw_ref, la_ref, lb_ref,
                 oh1_ref, oh2_ref, w1_ref, w2_ref, lora_ref, xpk_ref):
    x = x_ref[...]
    logits = jnp.dot(x, rw_ref[...], preferred_element_type=jnp.float32)
    m = jnp.max(logits, axis=-1, keepdims=True)
    p = jnp.exp(logits - m)
    p = p / jnp.sum(p, axis=-1, keepdims=True)
    eidx = jax.lax.broadcasted_iota(jnp.int32, p.shape, 1)
    i1 = jnp.argmax(p, axis=-1)
    oh1 = (eidx == i1[:, None]).astype(jnp.float32)
    v1 = jnp.max(p, axis=-1)
    pm = jnp.where(oh1 > 0, -jnp.inf, p)
    i2 = jnp.argmax(pm, axis=-1)
    oh2 = (eidx == i2[:, None]).astype(jnp.float32)
    v2 = jnp.max(pm, axis=-1)
    den = v1 + v2
    oh1_ref[...] = oh1
    oh2_ref[...] = oh2
    w1_ref[...] = (v1 / den)[:, None]
    w2_ref[...] = (v2 / den)[:, None]
    r = la_ref.shape[1]
    lora = jnp.dot(jnp.dot(x, la_ref[...], preferred_element_type=jnp.float32),
                   lb_ref[...], preferred_element_type=jnp.float32)
    lora_ref[...] = lora * (_ALPHA / r)
    hh = x.shape[1] // 2
    xpk_ref[...] = _pack_bf16(x[:, :hh], x[:, hh:])


def _pack_bf16(a, b):
    ua = lax.bitcast_convert_type(a.astype(jnp.bfloat16), jnp.uint16)
    ub = lax.bitcast_convert_type(b.astype(jnp.bfloat16), jnp.uint16)
    w = ua.astype(jnp.uint32) | (ub.astype(jnp.uint32) << 16)
    return lax.bitcast_convert_type(w, jnp.int32)


def _unpack_bf16(w):
    u = lax.bitcast_convert_type(w, jnp.uint32)
    lo = lax.bitcast_convert_type((u & 0xFFFF).astype(jnp.uint16), jnp.bfloat16)
    hi = lax.bitcast_convert_type((u >> 16).astype(jnp.uint16), jnp.bfloat16)
    return lo, hi


def _router(x, router_w, lora_A, lora_B):
    t, h = x.shape
    e = router_w.shape[1]
    r = lora_A.shape[1]
    tb = 512
    return pl.pallas_call(
        _router_body,
        grid=(t // tb,),
        in_specs=[
            pl.BlockSpec((tb, h), lambda i: (i, 0)),
            pl.BlockSpec((h, e), lambda i: (0, 0)),
            pl.BlockSpec((h, r), lambda i: (0, 0)),
            pl.BlockSpec((r, h), lambda i: (0, 0)),
        ],
        out_specs=[
            pl.BlockSpec((tb, e), lambda i: (i, 0)),
            pl.BlockSpec((tb, e), lambda i: (i, 0)),
            pl.BlockSpec((tb, 1), lambda i: (i, 0)),
            pl.BlockSpec((tb, 1), lambda i: (i, 0)),
            pl.BlockSpec((tb, h), lambda i: (i, 0)),
            pl.BlockSpec((tb, h // 2), lambda i: (i, 0)),
        ],
        out_shape=[
            jax.ShapeDtypeStruct((t, e), jnp.float32),
            jax.ShapeDtypeStruct((t, e), jnp.float32),
            jax.ShapeDtypeStruct((t, 1), jnp.float32),
            jax.ShapeDtypeStruct((t, 1), jnp.float32),
            jax.ShapeDtypeStruct((t, h), jnp.float32),
            jax.ShapeDtypeStruct((t, h // 2), jnp.int32),
        ],
    )(x, router_w, lora_A, lora_B)


def _dispatch_body(oh1_ref, oh2_ref, pos_ref, be_ref):
    t, e = oh1_ref.shape
    npairs = 2 * t
    nchunks = npairs // _BLK
    half = nchunks // 2
    nb_total = pos_ref.shape[0] // _BLK + e

    rows = jax.lax.broadcasted_iota(jnp.int32, (_BLK, _BLK), 0)
    cols = jax.lax.broadcasted_iota(jnp.int32, (_BLK, _BLK), 1)
    lts = (rows > cols).astype(jnp.float32)

    def load_chunk(j):
        o = (lax.rem(j, half)) * _BLK
        a = oh1_ref[pl.ds(o, _BLK), :]
        b = oh2_ref[pl.ds(o, _BLK), :]
        return jnp.where(j < half, a, b)

    cnt = (jnp.sum(oh1_ref[...], axis=0, keepdims=True)
           + jnp.sum(oh2_ref[...], axis=0, keepdims=True))

    nb = jnp.floor((cnt + (_BLK - 1)) * (1.0 / _BLK))
    ri = jax.lax.broadcasted_iota(jnp.int32, (e, e), 0)
    ci = jax.lax.broadcasted_iota(jnp.int32, (e, e), 1)
    ut = (ri < ci).astype(jnp.float32)
    start_blk = jnp.dot(nb, ut, preferred_element_type=jnp.float32)
    off = start_blk * float(_BLK)

    def pass2(j, carry):
        oh = load_chunk(j)
        rank = jnp.dot(lts, oh, preferred_element_type=jnp.float32) + carry
        posf = jnp.sum(oh * (rank + off), axis=1, keepdims=True)
        pos_ref[pl.ds(j * _BLK, _BLK), :] = posf.astype(jnp.int32)
        return carry + jnp.sum(oh, axis=0, keepdims=True)

    lax.fori_loop(0, nchunks, pass2, jnp.zeros((1, e), jnp.float32))

    bi = jax.lax.broadcasted_iota(jnp.int32, (nb_total, e), 0).astype(jnp.float32)
    be = jnp.sum((bi >= start_blk).astype(jnp.float32), axis=1, keepdims=True)
    be_ref[...] = be.astype(jnp.int32) - 1


def _dispatch(oh1, oh2, nb_total):
    t, e = oh1.shape
    return pl.pallas_call(
        _dispatch_body,
        out_shape=[
            jax.ShapeDtypeStruct((2 * t, 1), jnp.int32),
            jax.ShapeDtypeStruct((nb_total, 1), jnp.int32),
        ],
    )(oh1, oh2)


def _make_sc_scatter(t, hw, cap, nw):
    rows_per_w = 2 * t // nw
    nchunk = rows_per_w // _CH
    mesh = plsc.VectorSubcoreMesh(core_axis_name="c", subcore_axis_name="s",
                                  num_cores=2, num_subcores=16)

    @functools.partial(
        pl.kernel,
        out_type=jax.ShapeDtypeStruct((cap, hw), jnp.int32),
        mesh=mesh,
        scratch_types=[
            pltpu.VMEM((nchunk, _CH), jnp.int32),
            pltpu.VMEM((2, _CH, hw), jnp.int32),
            pltpu.SemaphoreType.DMA,
            pltpu.SemaphoreType.DMA,
            pltpu.SemaphoreType.DMA,
            pltpu.SemaphoreType.DMA,
        ],
    )
    def sc_scatter(pos2d, x, xs, idx_v, xbuf, lsem0, lsem1, ssem0, ssem1):
        nc = 2
        wid = lax.axis_index("s") * nc + lax.axis_index("c")
        row0 = wid * nchunk
        pltpu.sync_copy(pos2d.at[pl.ds(row0, nchunk)], idx_v)
        tbase = lax.rem(wid, nw // 2) * rows_per_w
        lsems = (lsem0, lsem1)
        ssems = (ssem0, ssem1)

        loads = [None] * nchunk
        stores = [None] * nchunk
        loads[0] = pltpu.async_copy(
            x.at[pl.ds(tbase, _CH)], xbuf.at[0], lsems[0])
        for c in range(nchunk):
            if c >= 1:
                stores[c - 1].wait()
            if c + 1 < nchunk:
                loads[c + 1] = pltpu.async_copy(
                    x.at[pl.ds(tbase + (c + 1) * _CH, _CH)],
                    xbuf.at[(c + 1) % 2], lsems[(c + 1) % 2])
            loads[c].wait()
            stores[c] = pltpu.async_copy(
                xbuf.at[c % 2], xs.at[idx_v.at[c]], ssems[c % 2])
        stores[nchunk - 1].wait()

    return sc_scatter


def _gmm_body(be_ref, xs_ref, wg_ref, wu_ref, wd_ref, out_ref):
    xlo, xhi = _unpack_bf16(xs_ref[...])
    hh = xlo.shape[1]
    wg = wg_ref[0].astype(jnp.bfloat16)
    wu = wu_ref[0].astype(jnp.bfloat16)
    wd = wd_ref[0].astype(jnp.bfloat16)
    g = (jnp.dot(xlo, wg[:hh], preferred_element_type=jnp.float32)
         + jnp.dot(xhi, wg[hh:], preferred_element_type=jnp.float32))
    u = (jnp.dot(xlo, wu[:hh], preferred_element_type=jnp.float32)
         + jnp.dot(xhi, wu[hh:], preferred_element_type=jnp.float32))
    hdn = (g * jax.nn.sigmoid(g)) * u
    y = jnp.dot(hdn.astype(jnp.bfloat16), wd,
                preferred_element_type=jnp.float32)
    hh = y.shape[1] // 2
    out_ref[...] = _pack_bf16(y[:, :hh], y[:, hh:])


def _gmm(xs, W_gate, W_up, W_down, be):
    cap, hw = xs.shape
    h = 2 * hw
    e, _, esz = W_gate.shape
    nb_total = cap // _BLK
    grid_spec = pltpu.PrefetchScalarGridSpec(
        num_scalar_prefetch=1,
        grid=(nb_total,),
        in_specs=[
            pl.BlockSpec((_BLK, hw), lambda i, s: (i, 0)),
            pl.BlockSpec((1, h, esz), lambda i, s: (s[i], 0, 0)),
            pl.BlockSpec((1, h, esz), lambda i, s: (s[i], 0, 0)),
            pl.BlockSpec((1, esz, h), lambda i, s: (s[i], 0, 0)),
        ],
        out_specs=pl.BlockSpec((_BLK, hw), lambda i, s: (i, 0)),
    )
    return pl.pallas_call(
        _gmm_body,
        grid_spec=grid_spec,
        out_shape=jax.ShapeDtypeStruct((cap, hw), jnp.int32),
        compiler_params=pltpu.CompilerParams(
            dimension_semantics=("arbitrary",),
        ),
    )(be, xs, W_gate, W_up, W_down)


def _make_sc_gather(t, hw, cap, nw):
    pairs_per_w = 2 * t // nw
    nchunk = pairs_per_w // _CH
    mesh = plsc.VectorSubcoreMesh(core_axis_name="c", subcore_axis_name="s",
                                  num_cores=2, num_subcores=16)

    @functools.partial(
        pl.kernel,
        out_type=jax.ShapeDtypeStruct((2 * t, hw), jnp.int32),
        mesh=mesh,
        scratch_types=[
            pltpu.VMEM((nchunk, _CH), jnp.int32),
            pltpu.VMEM((2, _CH, hw), jnp.int32),
            pltpu.SemaphoreType.DMA,
            pltpu.SemaphoreType.DMA,
            pltpu.SemaphoreType.DMA,
            pltpu.SemaphoreType.DMA,
        ],
    )
    def sc_gather(ys, pos2d, ytcat, idx_v, buf, gsem0, gsem1, ssem0, ssem1):
        nc = 2
        wid = lax.axis_index("s") * nc + lax.axis_index("c")
        pltpu.sync_copy(pos2d.at[pl.ds(wid * nchunk, nchunk)], idx_v)
        p0 = wid * pairs_per_w
        gsems = (gsem0, gsem1)
        ssems = (ssem0, ssem1)

        gathers = [None] * nchunk
        stores = [None] * nchunk
        gathers[0] = pltpu.async_copy(ys.at[idx_v.at[0]], buf.at[0], gsems[0])
        for c in range(nchunk):
            if c >= 1:
                stores[c - 1].wait()
            if c + 1 < nchunk:
                gathers[c + 1] = pltpu.async_copy(
                    ys.at[idx_v.at[c + 1]], buf.at[(c + 1) % 2],
                    gsems[(c + 1) % 2])
            gathers[c].wait()
            stores[c] = pltpu.async_copy(
                buf.at[c % 2], ytcat.at[pl.ds(p0 + c * _CH, _CH)],
                ssems[c % 2])
        stores[nchunk - 1].wait()

    return sc_gather


def _finalize_body(lora_ref, w1_ref, w2_ref, y1_ref, y2_ref, out_ref):
    y1lo, y1hi = _unpack_bf16(y1_ref[...])
    y1 = jnp.concatenate([y1lo, y1hi], axis=1).astype(jnp.float32)
    y2lo, y2hi = _unpack_bf16(y2_ref[...])
    y2 = jnp.concatenate([y2lo, y2hi], axis=1).astype(jnp.float32)
    out_ref[...] = (lora_ref[...]
                    + w1_ref[...] * y1
                    + w2_ref[...] * y2)


def _finalize(lora, w1, w2, ytcat):
    t, h = lora.shape
    tb = 512
    return pl.pallas_call(
        _finalize_body,
        grid=(t // tb,),
        in_specs=[
            pl.BlockSpec((tb, h), lambda i: (i, 0)),
            pl.BlockSpec((tb, 1), lambda i: (i, 0)),
            pl.BlockSpec((tb, 1), lambda i: (i, 0)),
            pl.BlockSpec((tb, h // 2), lambda i: (i, 0)),
            pl.BlockSpec((tb, h // 2), lambda i: (i + t // tb, 0)),
        ],
        out_specs=pl.BlockSpec((tb, h), lambda i: (i, 0)),
        out_shape=jax.ShapeDtypeStruct((t, h), jnp.float32),
    )(lora, w1, w2, ytcat, ytcat)


def kernel(input, router_w, W_gate, W_up, W_down, lora_A, lora_B):
    b, s, h = input.shape
    t = b * s
    e = router_w.shape[1]
    x = input.reshape(t, h)
    nb_total = (2 * t) // _BLK + e
    cap = nb_total * _BLK
    nw = 32

    hw = h // 2

    oh1, oh2, w1, w2, lora, xpk = _router(x, router_w, lora_A, lora_B)
    pos, be = _dispatch(oh1, oh2, nb_total)
    pos2d = pos.reshape(2 * t // _CH, _CH)

    xs = _make_sc_scatter(t, hw, cap, nw)(pos2d, xpk)
    ys = _gmm(xs, W_gate, W_up, W_down, be.reshape(nb_total))
    ytcat = _make_sc_gather(t, hw, cap, nw)(ys, pos2d)
    out = _finalize(lora, w1, w2, ytcat)
    return out.reshape(b, s, h)


# speedup vs baseline: 5.1925x; 1.0140x over previous
import functools

import jax
import jax.numpy as jnp
from jax import lax
from jax.experimental import pallas as pl
from jax.experimental.pallas import tpu as pltpu
from jax.experimental.pallas import tpu_sc as plsc

_ALPHA = 32.0
_BLK = 256
_CH = 64


def _pack_bf16(a, b):
    ua = lax.bitcast_convert_type(a.astype(jnp.bfloat16), jnp.uint16)
    ub = lax.bitcast_convert_type(b.astype(jnp.bfloat16), jnp.uint16)
    w = ua.astype(jnp.uint32) | (ub.astype(jnp.uint32) << 16)
    return lax.bitcast_convert_type(w, jnp.int32)


def _unpack_bf16(w):
    u = lax.bitcast_convert_type(w, jnp.uint32)
    lo = lax.bitcast_convert_type((u & 0xFFFF).astype(jnp.uint16), jnp.bfloat16)
    hi = lax.bitcast_convert_type((u >> 16).astype(jnp.uint16), jnp.bfloat16)
    return lo, hi


def _route_body(x_ref, rw_ref, pos_ref, be_ref, w1_ref, w2_ref, xpk_ref,
                oh1_ref, oh2_ref):
    t, h = x_ref.shape
    e = rw_ref.shape[1]
    nchunks = (2 * t) // _BLK
    half = nchunks // 2
    nb_total = pos_ref.shape[0] // _BLK + e

    x = x_ref[...]
    logits = jnp.dot(x, rw_ref[...], preferred_element_type=jnp.float32)
    m = jnp.max(logits, axis=-1, keepdims=True)
    p = jnp.exp(logits - m)
    p = p / jnp.sum(p, axis=-1, keepdims=True)
    eidx = jax.lax.broadcasted_iota(jnp.int32, p.shape, 1)
    i1 = jnp.argmax(p, axis=-1)
    oh1 = (eidx == i1[:, None]).astype(jnp.float32)
    v1 = jnp.max(p, axis=-1)
    pm = jnp.where(oh1 > 0, -jnp.inf, p)
    i2 = jnp.argmax(pm, axis=-1)
    oh2 = (eidx == i2[:, None]).astype(jnp.float32)
    v2 = jnp.max(pm, axis=-1)
    den = v1 + v2
    w1_ref[...] = (v1 / den)[:, None]
    w2_ref[...] = (v2 / den)[:, None]
    oh1_ref[...] = oh1
    oh2_ref[...] = oh2
    hh = h // 2
    xpk_ref[...] = _pack_bf16(x[:, :hh], x[:, hh:])

    rows = jax.lax.broadcasted_iota(jnp.int32, (_BLK, _BLK), 0)
    cols = jax.lax.broadcasted_iota(jnp.int32, (_BLK, _BLK), 1)
    lts = (rows > cols).astype(jnp.float32)

    cnt = (jnp.sum(oh1, axis=0, keepdims=True)
           + jnp.sum(oh2, axis=0, keepdims=True))
    nb = jnp.floor((cnt + (_BLK - 1)) * (1.0 / _BLK))
    ri = jax.lax.broadcasted_iota(jnp.int32, (e, e), 0)
    ci = jax.lax.broadcasted_iota(jnp.int32, (e, e), 1)
    ut = (ri < ci).astype(jnp.float32)
    start_blk = jnp.dot(nb, ut, preferred_element_type=jnp.float32)
    off = start_blk * float(_BLK)

    def load_chunk(j):
        o = (lax.rem(j, half)) * _BLK
        a = oh1_ref[pl.ds(o, _BLK), :]
        b = oh2_ref[pl.ds(o, _BLK), :]
        return jnp.where(j < half, a, b)

    def body(j, carry):
        oh = load_chunk(j)
        rank = jnp.dot(lts, oh, preferred_element_type=jnp.float32) + carry
        posf = jnp.sum(oh * (rank + off), axis=1, keepdims=True)
        pos_ref[pl.ds(j * _BLK, _BLK), :] = posf.astype(jnp.int32)
        return carry + jnp.sum(oh, axis=0, keepdims=True)

    lax.fori_loop(0, nchunks, body, jnp.zeros((1, e), jnp.float32))

    bi = jax.lax.broadcasted_iota(jnp.int32, (nb_total, e), 0).astype(jnp.float32)
    be = jnp.sum((bi >= start_blk).astype(jnp.float32), axis=1, keepdims=True)
    be_ref[...] = be.astype(jnp.int32) - 1


def _route(x, router_w, nb_total):
    t, h = x.shape
    e = router_w.shape[1]
    return pl.pallas_call(
        _route_body,
        out_shape=[
            jax.ShapeDtypeStruct((2 * t, 1), jnp.int32),
            jax.ShapeDtypeStruct((nb_total, 1), jnp.int32),
            jax.ShapeDtypeStruct((t, 1), jnp.float32),
            jax.ShapeDtypeStruct((t, 1), jnp.float32),
            jax.ShapeDtypeStruct((t, h // 2), jnp.int32),
        ],
        scratch_shapes=[
            pltpu.VMEM((t, e), jnp.float32),
            pltpu.VMEM((t, e), jnp.float32),
        ],
    )(x, router_w)


def _make_sc_scatter(t, hw, cap, nw):
    rows_per_w = 2 * t // nw
    nchunk = rows_per_w // _CH
    mesh = plsc.VectorSubcoreMesh(core_axis_name="c", subcore_axis_name="s",
                                  num_cores=2, num_subcores=16)

    @functools.partial(
        pl.kernel,
        out_type=jax.ShapeDtypeStruct((cap, hw), jnp.int32),
        mesh=mesh,
        scratch_types=[
            pltpu.VMEM((nchunk, _CH), jnp.int32),
            pltpu.VMEM((2, _CH, hw), jnp.int32),
            pltpu.SemaphoreType.DMA,
            pltpu.SemaphoreType.DMA,
            pltpu.SemaphoreType.DMA,
            pltpu.SemaphoreType.DMA,
        ],
    )
    def sc_scatter(pos2d, x, xs, idx_v, xbuf, lsem0, lsem1, ssem0, ssem1):
        nc = 2
        wid = lax.axis_index("s") * nc + lax.axis_index("c")
        row0 = wid * nchunk
        pltpu.sync_copy(pos2d.at[pl.ds(row0, nchunk)], idx_v)
        tbase = lax.rem(wid, nw // 2) * rows_per_w
        lsems = (lsem0, lsem1)
        ssems = (ssem0, ssem1)

        loads = [None] * nchunk
        stores = [None] * nchunk
        loads[0] = pltpu.async_copy(
            x.at[pl.ds(tbase, _CH)], xbuf.at[0], lsems[0])
        for c in range(nchunk):
            if c >= 1:
                stores[c - 1].wait()
            if c + 1 < nchunk:
                loads[c + 1] = pltpu.async_copy(
                    x.at[pl.ds(tbase + (c + 1) * _CH, _CH)],
                    xbuf.at[(c + 1) % 2], lsems[(c + 1) % 2])
            loads[c].wait()
            stores[c] = pltpu.async_copy(
                xbuf.at[c % 2], xs.at[idx_v.at[c]], ssems[c % 2])
        stores[nchunk - 1].wait()

    return sc_scatter


def _gmm_body(be_ref, xs_ref, wg_ref, wu_ref, wd_ref, la_ref, lb_ref,
              out_ref):
    xlo, xhi = _unpack_bf16(xs_ref[...])
    hh = xlo.shape[1]
    wg = wg_ref[0].astype(jnp.bfloat16)
    wu = wu_ref[0].astype(jnp.bfloat16)
    wd = wd_ref[0].astype(jnp.bfloat16)
    g = (jnp.dot(xlo, wg[:hh], preferred_element_type=jnp.float32)
         + jnp.dot(xhi, wg[hh:], preferred_element_type=jnp.float32))
    u = (jnp.dot(xlo, wu[:hh], preferred_element_type=jnp.float32)
         + jnp.dot(xhi, wu[hh:], preferred_element_type=jnp.float32))
    hdn = (g * jax.nn.sigmoid(g)) * u
    y = jnp.dot(hdn.astype(jnp.bfloat16), wd,
                preferred_element_type=jnp.float32)
    r = la_ref.shape[1]
    la = la_ref[...].astype(jnp.bfloat16)
    lb = lb_ref[...].astype(jnp.bfloat16)
    xa = (jnp.dot(xlo, la[:hh], preferred_element_type=jnp.float32)
          + jnp.dot(xhi, la[hh:], preferred_element_type=jnp.float32))
    y = y + jnp.dot(xa.astype(jnp.bfloat16), lb,
                    preferred_element_type=jnp.float32) * (_ALPHA / r)
    yh = y.shape[1] // 2
    out_ref[...] = _pack_bf16(y[:, :yh], y[:, yh:])


def _gmm(xs, W_gate, W_up, W_down, lora_A, lora_B, be):
    cap, hw = xs.shape
    h = 2 * hw
    e, _, esz = W_gate.shape
    r = lora_A.shape[1]
    nb_total = cap // _BLK
    grid_spec = pltpu.PrefetchScalarGridSpec(
        num_scalar_prefetch=1,
        grid=(nb_total,),
        in_specs=[
            pl.BlockSpec((_BLK, hw), lambda i, s: (i, 0)),
            pl.BlockSpec((1, h, esz), lambda i, s: (s[i], 0, 0)),
            pl.BlockSpec((1, h, esz), lambda i, s: (s[i], 0, 0)),
            pl.BlockSpec((1, esz, h), lambda i, s: (s[i], 0, 0)),
            pl.BlockSpec((h, r), lambda i, s: (0, 0)),
            pl.BlockSpec((r, h), lambda i, s: (0, 0)),
        ],
        out_specs=pl.BlockSpec((_BLK, hw), lambda i, s: (i, 0)),
    )
    return pl.pallas_call(
        _gmm_body,
        grid_spec=grid_spec,
        out_shape=jax.ShapeDtypeStruct((cap, hw), jnp.int32),
        compiler_params=pltpu.CompilerParams(
            dimension_semantics=("arbitrary",),
        ),
    )(be, xs, W_gate, W_up, W_down, lora_A, lora_B)


def _make_sc_gather(t, hw, cap, nw):
    pairs_per_w = 2 * t // nw
    nchunk = pairs_per_w // _CH
    mesh = plsc.VectorSubcoreMesh(core_axis_name="c", subcore_axis_name="s",
                                  num_cores=2, num_subcores=16)

    @functools.partial(
        pl.kernel,
        out_type=jax.ShapeDtypeStruct((2 * t, hw), jnp.int32),
        mesh=mesh,
        scratch_types=[
            pltpu.VMEM((nchunk, _CH), jnp.int32),
            pltpu.VMEM((2, _CH, hw), jnp.int32),
            pltpu.SemaphoreType.DMA,
            pltpu.SemaphoreType.DMA,
            pltpu.SemaphoreType.DMA,
            pltpu.SemaphoreType.DMA,
        ],
    )
    def sc_gather(ys, pos2d, ytcat, idx_v, buf, gsem0, gsem1, ssem0, ssem1):
        nc = 2
        wid = lax.axis_index("s") * nc + lax.axis_index("c")
        pltpu.sync_copy(pos2d.at[pl.ds(wid * nchunk, nchunk)], idx_v)
        p0 = wid * pairs_per_w
        gsems = (gsem0, gsem1)
        ssems = (ssem0, ssem1)

        gathers = [None] * nchunk
        stores = [None] * nchunk
        gathers[0] = pltpu.async_copy(ys.at[idx_v.at[0]], buf.at[0], gsems[0])
        for c in range(nchunk):
            if c >= 1:
                stores[c - 1].wait()
            if c + 1 < nchunk:
                gathers[c + 1] = pltpu.async_copy(
                    ys.at[idx_v.at[c + 1]], buf.at[(c + 1) % 2],
                    gsems[(c + 1) % 2])
            gathers[c].wait()
            stores[c] = pltpu.async_copy(
                buf.at[c % 2], ytcat.at[pl.ds(p0 + c * _CH, _CH)],
                ssems[c % 2])
        stores[nchunk - 1].wait()

    return sc_gather


def _finalize_body(w1_ref, w2_ref, y1_ref, y2_ref, out_ref):
    y1lo, y1hi = _unpack_bf16(y1_ref[...])
    y1 = jnp.concatenate([y1lo, y1hi], axis=1).astype(jnp.float32)
    y2lo, y2hi = _unpack_bf16(y2_ref[...])
    y2 = jnp.concatenate([y2lo, y2hi], axis=1).astype(jnp.float32)
    out_ref[...] = w1_ref[...] * y1 + w2_ref[...] * y2


def _finalize(w1, w2, ytcat, h):
    t = w1.shape[0]
    tb = 512
    return pl.pallas_call(
        _finalize_body,
        grid=(t // tb,),
        in_specs=[
            pl.BlockSpec((tb, 1), lambda i: (i, 0)),
            pl.BlockSpec((tb, 1), lambda i: (i, 0)),
            pl.BlockSpec((tb, h // 2), lambda i: (i, 0)),
            pl.BlockSpec((tb, h // 2), lambda i: (i + t // tb, 0)),
        ],
        out_specs=pl.BlockSpec((tb, h), lambda i: (i, 0)),
        out_shape=jax.ShapeDtypeStruct((t, h), jnp.float32),
    )(w1, w2, ytcat, ytcat)


def kernel(input, router_w, W_gate, W_up, W_down, lora_A, lora_B):
    b, s, h = input.shape
    t = b * s
    e = router_w.shape[1]
    x = input.reshape(t, h)
    nb_total = (2 * t) // _BLK + e
    cap = nb_total * _BLK
    nw = 32
    hw = h // 2

    pos, be, w1, w2, xpk = _route(x, router_w, nb_total)
    pos2d = pos.reshape(2 * t // _CH, _CH)

    xs = _make_sc_scatter(t, hw, cap, nw)(pos2d, xpk)
    ys = _gmm(xs, W_gate, W_up, W_down, lora_A, lora_B,
              be.reshape(nb_total))
    ytcat = _make_sc_gather(t, hw, cap, nw)(ys, pos2d)
    out = _finalize(w1, w2, ytcat, h)
    return out.reshape(b, s, h)


# speedup vs baseline: 5.3449x; 1.0294x over previous
import functools

import jax
import jax.numpy as jnp
from jax import lax
from jax.experimental import pallas as pl
from jax.experimental.pallas import tpu as pltpu
from jax.experimental.pallas import tpu_sc as plsc

_ALPHA = 32.0
_BLK = 512
_CH = 64


def _pack_bf16(a, b):
    ua = lax.bitcast_convert_type(a.astype(jnp.bfloat16), jnp.uint16)
    ub = lax.bitcast_convert_type(b.astype(jnp.bfloat16), jnp.uint16)
    w = ua.astype(jnp.uint32) | (ub.astype(jnp.uint32) << 16)
    return lax.bitcast_convert_type(w, jnp.int32)


def _unpack_bf16(w):
    u = lax.bitcast_convert_type(w, jnp.uint32)
    lo = lax.bitcast_convert_type((u & 0xFFFF).astype(jnp.uint16), jnp.bfloat16)
    hi = lax.bitcast_convert_type((u >> 16).astype(jnp.uint16), jnp.bfloat16)
    return lo, hi


def _route_body(x_ref, rw_ref, pos_ref, be_ref, w1_ref, w2_ref, xpk_ref,
                oh1_ref, oh2_ref):
    t, h = x_ref.shape
    e = rw_ref.shape[1]
    nchunks = (2 * t) // _BLK
    half = nchunks // 2
    nb_total = pos_ref.shape[0] // _BLK + e

    x = x_ref[...]
    logits = jnp.dot(x, rw_ref[...], preferred_element_type=jnp.float32)
    m = jnp.max(logits, axis=-1, keepdims=True)
    p = jnp.exp(logits - m)
    p = p / jnp.sum(p, axis=-1, keepdims=True)
    eidx = jax.lax.broadcasted_iota(jnp.int32, p.shape, 1)
    i1 = jnp.argmax(p, axis=-1)
    oh1 = (eidx == i1[:, None]).astype(jnp.float32)
    v1 = jnp.max(p, axis=-1)
    pm = jnp.where(oh1 > 0, -jnp.inf, p)
    i2 = jnp.argmax(pm, axis=-1)
    oh2 = (eidx == i2[:, None]).astype(jnp.float32)
    v2 = jnp.max(pm, axis=-1)
    den = v1 + v2
    w1_ref[...] = (v1 / den)[:, None]
    w2_ref[...] = (v2 / den)[:, None]
    oh1_ref[...] = oh1
    oh2_ref[...] = oh2
    hh = h // 2
    xpk_ref[...] = _pack_bf16(x[:, :hh], x[:, hh:])

    rows = jax.lax.broadcasted_iota(jnp.int32, (_BLK, _BLK), 0)
    cols = jax.lax.broadcasted_iota(jnp.int32, (_BLK, _BLK), 1)
    lts = (rows > cols).astype(jnp.float32)

    cnt = (jnp.sum(oh1, axis=0, keepdims=True)
           + jnp.sum(oh2, axis=0, keepdims=True))
    nb = jnp.floor((cnt + (_BLK - 1)) * (1.0 / _BLK))
    ri = jax.lax.broadcasted_iota(jnp.int32, (e, e), 0)
    ci = jax.lax.broadcasted_iota(jnp.int32, (e, e), 1)
    ut = (ri < ci).astype(jnp.float32)
    start_blk = jnp.dot(nb, ut, preferred_element_type=jnp.float32)
    off = start_blk * float(_BLK)

    def load_chunk(j):
        o = (lax.rem(j, half)) * _BLK
        a = oh1_ref[pl.ds(o, _BLK), :]
        b = oh2_ref[pl.ds(o, _BLK), :]
        return jnp.where(j < half, a, b)

    def body(j, carry):
        oh = load_chunk(j)
        rank = jnp.dot(lts, oh, preferred_element_type=jnp.float32) + carry
        posf = jnp.sum(oh * (rank + off), axis=1, keepdims=True)
        pos_ref[pl.ds(j * _BLK, _BLK), :] = posf.astype(jnp.int32)
        return carry + jnp.sum(oh, axis=0, keepdims=True)

    lax.fori_loop(0, nchunks, body, jnp.zeros((1, e), jnp.float32))

    bi = jax.lax.broadcasted_iota(jnp.int32, (nb_total, e), 0).astype(jnp.float32)
    be = jnp.sum((bi >= start_blk).astype(jnp.float32), axis=1, keepdims=True)
    be_ref[...] = be.astype(jnp.int32) - 1


def _route(x, router_w, nb_total):
    t, h = x.shape
    e = router_w.shape[1]
    return pl.pallas_call(
        _route_body,
        out_shape=[
            jax.ShapeDtypeStruct((2 * t, 1), jnp.int32),
            jax.ShapeDtypeStruct((nb_total, 1), jnp.int32),
            jax.ShapeDtypeStruct((t, 1), jnp.float32),
            jax.ShapeDtypeStruct((t, 1), jnp.float32),
            jax.ShapeDtypeStruct((t, h // 2), jnp.int32),
        ],
        scratch_shapes=[
            pltpu.VMEM((t, e), jnp.float32),
            pltpu.VMEM((t, e), jnp.float32),
        ],
    )(x, router_w)


def _make_sc_scatter(t, hw, cap, nw):
    rows_per_w = 2 * t // nw
    nchunk = rows_per_w // _CH
    mesh = plsc.VectorSubcoreMesh(core_axis_name="c", subcore_axis_name="s",
                                  num_cores=2, num_subcores=16)

    @functools.partial(
        pl.kernel,
        out_type=jax.ShapeDtypeStruct((cap, hw), jnp.int32),
        mesh=mesh,
        scratch_types=[
            pltpu.VMEM((nchunk, _CH), jnp.int32),
            pltpu.VMEM((2, _CH, hw), jnp.int32),
            pltpu.SemaphoreType.DMA,
            pltpu.SemaphoreType.DMA,
            pltpu.SemaphoreType.DMA,
            pltpu.SemaphoreType.DMA,
        ],
    )
    def sc_scatter(pos2d, x, xs, idx_v, xbuf, lsem0, lsem1, ssem0, ssem1):
        nc = 2
        wid = lax.axis_index("s") * nc + lax.axis_index("c")
        row0 = wid * nchunk
        pltpu.sync_copy(pos2d.at[pl.ds(row0, nchunk)], idx_v)
        tbase = lax.rem(wid, nw // 2) * rows_per_w
        lsems = (lsem0, lsem1)
        ssems = (ssem0, ssem1)

        loads = [None] * nchunk
        stores = [None] * nchunk
        loads[0] = pltpu.async_copy(
            x.at[pl.ds(tbase, _CH)], xbuf.at[0], lsems[0])
        for c in range(nchunk):
            if c >= 1:
                stores[c - 1].wait()
            if c + 1 < nchunk:
                loads[c + 1] = pltpu.async_copy(
                    x.at[pl.ds(tbase + (c + 1) * _CH, _CH)],
                    xbuf.at[(c + 1) % 2], lsems[(c + 1) % 2])
            loads[c].wait()
            stores[c] = pltpu.async_copy(
                xbuf.at[c % 2], xs.at[idx_v.at[c]], ssems[c % 2])
        stores[nchunk - 1].wait()

    return sc_scatter


def _gmm_body(be_ref, xs_ref, wg_ref, wu_ref, wd_ref, la_ref, lb_ref,
              out_ref):
    xlo, xhi = _unpack_bf16(xs_ref[...])
    hh = xlo.shape[1]
    wg = wg_ref[0].astype(jnp.bfloat16)
    wu = wu_ref[0].astype(jnp.bfloat16)
    wd = wd_ref[0].astype(jnp.bfloat16)
    g = (jnp.dot(xlo, wg[:hh], preferred_element_type=jnp.float32)
         + jnp.dot(xhi, wg[hh:], preferred_element_type=jnp.float32))
    u = (jnp.dot(xlo, wu[:hh], preferred_element_type=jnp.float32)
         + jnp.dot(xhi, wu[hh:], preferred_element_type=jnp.float32))
    hdn = (g * jax.nn.sigmoid(g)) * u
    y = jnp.dot(hdn.astype(jnp.bfloat16), wd,
                preferred_element_type=jnp.float32)
    r = la_ref.shape[1]
    la = la_ref[...].astype(jnp.bfloat16)
    lb = lb_ref[...].astype(jnp.bfloat16)
    xa = (jnp.dot(xlo, la[:hh], preferred_element_type=jnp.float32)
          + jnp.dot(xhi, la[hh:], preferred_element_type=jnp.float32))
    y = y + jnp.dot(xa.astype(jnp.bfloat16), lb,
                    preferred_element_type=jnp.float32) * (_ALPHA / r)
    yh = y.shape[1] // 2
    out_ref[...] = _pack_bf16(y[:, :yh], y[:, yh:])


def _gmm(xs, W_gate, W_up, W_down, lora_A, lora_B, be):
    cap, hw = xs.shape
    h = 2 * hw
    e, _, esz = W_gate.shape
    r = lora_A.shape[1]
    nb_total = cap // _BLK
    grid_spec = pltpu.PrefetchScalarGridSpec(
        num_scalar_prefetch=1,
        grid=(nb_total,),
        in_specs=[
            pl.BlockSpec((_BLK, hw), lambda i, s: (i, 0)),
            pl.BlockSpec((1, h, esz), lambda i, s: (s[i], 0, 0)),
            pl.BlockSpec((1, h, esz), lambda i, s: (s[i], 0, 0)),
            pl.BlockSpec((1, esz, h), lambda i, s: (s[i], 0, 0)),
            pl.BlockSpec((h, r), lambda i, s: (0, 0)),
            pl.BlockSpec((r, h), lambda i, s: (0, 0)),
        ],
        out_specs=pl.BlockSpec((_BLK, hw), lambda i, s: (i, 0)),
    )
    return pl.pallas_call(
        _gmm_body,
        grid_spec=grid_spec,
        out_shape=jax.ShapeDtypeStruct((cap, hw), jnp.int32),
        compiler_params=pltpu.CompilerParams(
            dimension_semantics=("arbitrary",),
        ),
    )(be, xs, W_gate, W_up, W_down, lora_A, lora_B)


def _make_sc_gather(t, hw, cap, nw):
    pairs_per_w = 2 * t // nw
    nchunk = pairs_per_w // _CH
    mesh = plsc.VectorSubcoreMesh(core_axis_name="c", subcore_axis_name="s",
                                  num_cores=2, num_subcores=16)

    @functools.partial(
        pl.kernel,
        out_type=jax.ShapeDtypeStruct((2 * t, hw), jnp.int32),
        mesh=mesh,
        scratch_types=[
            pltpu.VMEM((nchunk, _CH), jnp.int32),
            pltpu.VMEM((2, _CH, hw), jnp.int32),
            pltpu.SemaphoreType.DMA,
            pltpu.SemaphoreType.DMA,
            pltpu.SemaphoreType.DMA,
            pltpu.SemaphoreType.DMA,
        ],
    )
    def sc_gather(ys, pos2d, ytcat, idx_v, buf, gsem0, gsem1, ssem0, ssem1):
        nc = 2
        wid = lax.axis_index("s") * nc + lax.axis_index("c")
        pltpu.sync_copy(pos2d.at[pl.ds(wid * nchunk, nchunk)], idx_v)
        p0 = wid * pairs_per_w
        gsems = (gsem0, gsem1)
        ssems = (ssem0, ssem1)

        gathers = [None] * nchunk
        stores = [None] * nchunk
        gathers[0] = pltpu.async_copy(ys.at[idx_v.at[0]], buf.at[0], gsems[0])
        for c in range(nchunk):
            if c >= 1:
                stores[c - 1].wait()
            if c + 1 < nchunk:
                gathers[c + 1] = pltpu.async_copy(
                    ys.at[idx_v.at[c + 1]], buf.at[(c + 1) % 2],
                    gsems[(c + 1) % 2])
            gathers[c].wait()
            stores[c] = pltpu.async_copy(
                buf.at[c % 2], ytcat.at[pl.ds(p0 + c * _CH, _CH)],
                ssems[c % 2])
        stores[nchunk - 1].wait()

    return sc_gather


def _finalize_body(w1_ref, w2_ref, y1_ref, y2_ref, out_ref):
    y1lo, y1hi = _unpack_bf16(y1_ref[...])
    y1 = jnp.concatenate([y1lo, y1hi], axis=1).astype(jnp.float32)
    y2lo, y2hi = _unpack_bf16(y2_ref[...])
    y2 = jnp.concatenate([y2lo, y2hi], axis=1).astype(jnp.float32)
    out_ref[...] = w1_ref[...] * y1 + w2_ref[...] * y2


def _finalize(w1, w2, ytcat, h):
    t = w1.shape[0]
    tb = 512
    return pl.pallas_call(
        _finalize_body,
        grid=(t // tb,),
        in_specs=[
            pl.BlockSpec((tb, 1), lambda i: (i, 0)),
            pl.BlockSpec((tb, 1), lambda i: (i, 0)),
            pl.BlockSpec((tb, h // 2), lambda i: (i, 0)),
            pl.BlockSpec((tb, h // 2), lambda i: (i + t // tb, 0)),
        ],
        out_specs=pl.BlockSpec((tb, h), lambda i: (i, 0)),
        out_shape=jax.ShapeDtypeStruct((t, h), jnp.float32),
    )(w1, w2, ytcat, ytcat)


def kernel(input, router_w, W_gate, W_up, W_down, lora_A, lora_B):
    b, s, h = input.shape
    t = b * s
    e = router_w.shape[1]
    x = input.reshape(t, h)
    nb_total = (2 * t) // _BLK + e
    cap = nb_total * _BLK
    nw = 32
    hw = h // 2

    pos, be, w1, w2, xpk = _route(x, router_w, nb_total)
    pos2d = pos.reshape(2 * t // _CH, _CH)

    xs = _make_sc_scatter(t, hw, cap, nw)(pos2d, xpk)
    ys = _gmm(xs, W_gate, W_up, W_down, lora_A, lora_B,
              be.reshape(nb_total))
    ytcat = _make_sc_gather(t, hw, cap, nw)(ys, pos2d)
    out = _finalize(w1, w2, ytcat, h)
    return out.reshape(b, s, h)
